# baseline (speedup 1.0000x reference)
import jax
import jax.numpy as jnp
from jax.experimental import pallas as pl
from jax.experimental.pallas import tpu as pltpu

BATCH = 128
IN = 784
HID = 4096
H2 = 2048
H3 = 1024
OUT = 10

BK = 512
NK1 = HID // BK
NK2 = H2 // BK
NJ = H2 // BK
_S_END = NK1 + NK2

_NT = (((1,), (1,)), ((), ()))


def _dot_nt(a, b, fast=False):
    prec = jax.lax.Precision.DEFAULT if fast else jax.lax.Precision.HIGHEST
    return jax.lax.dot_general(a, b, _NT, precision=prec,
                               preferred_element_type=jnp.float32)


def _mega_kernel(x_ref, W0_ref, b0_ref, Wsel_ref, bsel_ref, Wsg_ref, bsg_ref,
                 Wsgo_ref, bsgo_ref, sl_ref,
                 W1_ref, W2_ref, W3_ref, b1_ref, b2_ref, b3_ref,
                 W11_ref, W22_ref, W33_ref, b11_ref, b22_ref, b33_ref,
                 Wout_ref, bout_ref, tgt_ref,
                 out_ref, loss_ref, synloss_ref,
                 sel_scr, idx_scr, a1_scr, a2_scr, a3_scr,
                 p1_scr, p2_scr, p3_scr):
    s = pl.program_id(0)

    @pl.when(s == 0)
    def _():
        sel_scr[:] = jnp.zeros_like(sel_scr)
        a1_scr[:] = jnp.zeros_like(a1_scr)
        a2_scr[:] = jnp.zeros_like(a2_scr)
        a3_scr[:] = jnp.zeros_like(a3_scr)
        p1_scr[:] = jnp.zeros_like(p1_scr)
        p2_scr[:] = jnp.zeros_like(p2_scr)
        p3_scr[:] = jnp.zeros_like(p3_scr)

    @pl.when(s < NK1)
    def _():
        m0 = _dot_nt(x_ref[:], W0_ref[:])
        hblk = jnp.maximum(m0 + b0_ref[:], 0.0)
        sel_scr[:] = sel_scr[:] + _dot_nt(hblk, Wsel_ref[:])
        c1 = _dot_nt(hblk, W1_ref[:], fast=True)
        c2 = _dot_nt(hblk, W2_ref[:], fast=True)
        c3 = _dot_nt(hblk, W3_ref[:], fast=True)
        for j in range(NJ):
            lo, hi = j * BK, (j + 1) * BK
            a1_scr[j] = a1_scr[j] + c1[:, lo:hi]
            a2_scr[j] = a2_scr[j] + c2[:, lo:hi]
            a3_scr[j] = a3_scr[j] + c3[:, lo:hi]

    @pl.when(s == NK1 - 1)
    def _():
        for j in range(NJ):
            lo, hi = j * BK, (j + 1) * BK
            a1_scr[j] = jnp.maximum(a1_scr[j] + b1_ref[:, lo:hi], 0.0)
            a2_scr[j] = jnp.maximum(a2_scr[j] + b2_ref[:, lo:hi], 0.0)
            a3_scr[j] = jnp.maximum(a3_scr[j] + b3_ref[:, lo:hi], 0.0)
        logits = sel_scr[:] + bsel_ref[:]
        m = jnp.max(logits, axis=1, keepdims=True)
        e = jnp.exp(logits - m)
        p = e / jnp.sum(e, axis=1, keepdims=True)
        syn = jax.nn.sigmoid(jnp.sum(p * Wsg_ref[:], axis=1, keepdims=True)
                             + bsg_ref[:])
        s2 = jax.nn.sigmoid(jnp.sum(syn * Wsgo_ref[:], axis=0, keepdims=True)
                            + bsgo_ref[:])
        synloss_ref[:] = (s2 - sl_ref[:]) ** 2
        p0 = p[:, 0:1]
        p1 = p[:, 1:2]
        p2 = p[:, 2:3]
        idx_scr[:] = jnp.where((p0 >= p1) & (p0 >= p2), 0.0,
                               jnp.where(p1 >= p2, 1.0, 2.0))

    @pl.when(s >= NK1)
    def _():
        j = s - NK1
        p1_scr[:] = p1_scr[:] + _dot_nt(a1_scr[j], W11_ref[:], fast=True)
        p2_scr[:] = p2_scr[:] + _dot_nt(a2_scr[j], W22_ref[:], fast=True)
        p3_scr[:] = p3_scr[:] + _dot_nt(a3_scr[j], W33_ref[:], fast=True)

    @pl.when(s == _S_END - 1)
    def _():
        f1 = jnp.maximum(p1_scr[:] + b11_ref[:], 0.0)
        f2 = jnp.maximum(p2_scr[:] + b22_ref[:], 0.0)
        f3 = jnp.maximum(p3_scr[:] + b33_ref[:], 0.0)
        idx = idx_scr[:]
        routed = jnp.where(idx == 0.0, f1,
                           jnp.where(idx == 1.0, f2, f3))
        o = _dot_nt(routed, Wout_ref[:])
        o = jnp.maximum(o + bout_ref[:], 0.0)
        out_ref[:] = o
        m = jnp.max(o, axis=1, keepdims=True)
        lse = jnp.log(jnp.sum(jnp.exp(o - m), axis=1, keepdims=True)) + m
        logp = o - lse
        cols = jax.lax.broadcasted_iota(jnp.int32, (BATCH, OUT), 1)
        oh = (cols == tgt_ref[:]).astype(jnp.float32)
        per_row = jnp.sum(logp * oh, axis=1, keepdims=True)
        loss_ref[:] = -jnp.sum(per_row, axis=0, keepdims=True) / BATCH


def kernel(x, target, selector_loss, W0, b0, Wsel, bsel, Wsg, bsg, Wsgo, bsgo,
           W1, b1, W11, b11, W2, b2, W22, b22, W3, b3, W33, b33, Wout, bout):
    x = x.reshape(-1, IN)
    tgt = target.reshape(BATCH, 1).astype(jnp.int32)

    const2 = lambda shp: pl.BlockSpec(shp, lambda s: (0, 0))
    w0spec = pl.BlockSpec((BK, IN), lambda s: (jnp.minimum(s, NK1 - 1), 0))
    b0spec = pl.BlockSpec((1, BK), lambda s: (0, jnp.minimum(s, NK1 - 1)))
    wselspec = pl.BlockSpec((3, BK), lambda s: (0, jnp.minimum(s, NK1 - 1)))
    w1spec = pl.BlockSpec((H2, BK), lambda s: (0, jnp.minimum(s, NK1 - 1)))
    w2spec = pl.BlockSpec(
        (H3, BK), lambda s: (0, jnp.clip(s - NK1, 0, NK2 - 1)))

    out, loss, synloss = pl.pallas_call(
        _mega_kernel,
        grid=(_S_END,),
        compiler_params=pltpu.CompilerParams(
            dimension_semantics=("arbitrary",)),
        in_specs=[
            const2((BATCH, IN)),
            w0spec,
            b0spec,
            wselspec,
            const2((1, 3)),
            const2((1, 3)),
            const2((1, 1)),
            const2((BATCH, 1)),
            const2((1, 1)),
            const2((1, 1)),
            w1spec, w1spec, w1spec,
            const2((1, H2)), const2((1, H2)), const2((1, H2)),
            w2spec, w2spec, w2spec,
            const2((1, H3)), const2((1, H3)), const2((1, H3)),
            const2((OUT, H3)),
            const2((1, OUT)),
            const2((BATCH, 1)),
        ],
        out_specs=[
            const2((BATCH, OUT)),
            const2((1, 1)),
            const2((1, 1)),
        ],
        out_shape=[
            jax.ShapeDtypeStruct((BATCH, OUT), jnp.float32),
            jax.ShapeDtypeStruct((1, 1), jnp.float32),
            jax.ShapeDtypeStruct((1, 1), jnp.float32),
        ],
        scratch_shapes=[
            pltpu.VMEM((BATCH, 3), jnp.float32),
            pltpu.VMEM((BATCH, 1), jnp.float32),
            pltpu.VMEM((NJ, BATCH, BK), jnp.float32),
            pltpu.VMEM((NJ, BATCH, BK), jnp.float32),
            pltpu.VMEM((NJ, BATCH, BK), jnp.float32),
            pltpu.VMEM((BATCH, H3), jnp.float32),
            pltpu.VMEM((BATCH, H3), jnp.float32),
            pltpu.VMEM((BATCH, H3), jnp.float32),
        ],
    )(x, W0, b0.reshape(1, HID), Wsel, bsel.reshape(1, 3), Wsg,
      bsg.reshape(1, 1), Wsgo.reshape(BATCH, 1), bsgo.reshape(1, 1),
      selector_loss.reshape(1, 1),
      W1, W2, W3, b1.reshape(1, H2), b2.reshape(1, H2), b3.reshape(1, H2),
      W11, W22, W33, b11.reshape(1, H3), b22.reshape(1, H3),
      b33.reshape(1, H3), Wout, bout.reshape(1, OUT), tgt)

    return (out, loss[0, 0], synloss[0, 0])


# speedup vs baseline: 1.0764x; 1.0764x over previous
import jax
import jax.numpy as jnp
from jax.experimental import pallas as pl
from jax.experimental.pallas import tpu as pltpu

BATCH = 128
IN = 784
HID = 4096
H2 = 2048
H3 = 1024
OUT = 10

BK = 512
NK1 = HID // BK
NK2 = H2 // BK
NJ = H2 // BK
_S_END = NK1 + NK2

_NT = (((1,), (1,)), ((), ()))


def _dot_nt(a, b):
    return jax.lax.dot_general(a, b, _NT, preferred_element_type=jnp.float32)


def _mega_kernel(x_ref, W0_ref, b0_ref, Wsel_ref, bsel_ref, Wsg_ref, bsg_ref,
                 Wsgo_ref, bsgo_ref, sl_ref,
                 W1_ref, W2_ref, W3_ref, b1_ref, b2_ref, b3_ref,
                 W11_ref, W22_ref, W33_ref, b11_ref, b22_ref, b33_ref,
                 Wout_ref, bout_ref, tgt_ref,
                 out_ref, loss_ref, synloss_ref,
                 sel_scr, idx_scr, a1_scr, a2_scr, a3_scr,
                 p1_scr, p2_scr, p3_scr):
    s = pl.program_id(0)

    @pl.when(s == 0)
    def _():
        sel_scr[:] = jnp.zeros_like(sel_scr)
        a1_scr[:] = jnp.zeros_like(a1_scr)
        a2_scr[:] = jnp.zeros_like(a2_scr)
        a3_scr[:] = jnp.zeros_like(a3_scr)
        p1_scr[:] = jnp.zeros_like(p1_scr)
        p2_scr[:] = jnp.zeros_like(p2_scr)
        p3_scr[:] = jnp.zeros_like(p3_scr)

    @pl.when(s < NK1)
    def _():
        m0 = _dot_nt(x_ref[:], W0_ref[:])
        hblk = jnp.maximum(m0 + b0_ref[:], 0.0)
        sel_scr[:] = sel_scr[:] + _dot_nt(hblk, Wsel_ref[:])
        c1 = _dot_nt(hblk, W1_ref[:])
        c2 = _dot_nt(hblk, W2_ref[:])
        c3 = _dot_nt(hblk, W3_ref[:])
        for j in range(NJ):
            lo, hi = j * BK, (j + 1) * BK
            a1_scr[j] = a1_scr[j] + c1[:, lo:hi]
            a2_scr[j] = a2_scr[j] + c2[:, lo:hi]
            a3_scr[j] = a3_scr[j] + c3[:, lo:hi]

    @pl.when(s == NK1 - 1)
    def _():
        for j in range(NJ):
            lo, hi = j * BK, (j + 1) * BK
            a1_scr[j] = jnp.maximum(a1_scr[j] + b1_ref[:, lo:hi], 0.0)
            a2_scr[j] = jnp.maximum(a2_scr[j] + b2_ref[:, lo:hi], 0.0)
            a3_scr[j] = jnp.maximum(a3_scr[j] + b3_ref[:, lo:hi], 0.0)
        logits = sel_scr[:] + bsel_ref[:]
        m = jnp.max(logits, axis=1, keepdims=True)
        e = jnp.exp(logits - m)
        p = e / jnp.sum(e, axis=1, keepdims=True)
        syn = jax.nn.sigmoid(jnp.sum(p * Wsg_ref[:], axis=1, keepdims=True)
                             + bsg_ref[:])
        s2 = jax.nn.sigmoid(jnp.sum(syn * Wsgo_ref[:], axis=0, keepdims=True)
                            + bsgo_ref[:])
        synloss_ref[:] = (s2 - sl_ref[:]) ** 2
        p0 = p[:, 0:1]
        p1 = p[:, 1:2]
        p2 = p[:, 2:3]
        idx_scr[:] = jnp.where((p0 >= p1) & (p0 >= p2), 0.0,
                               jnp.where(p1 >= p2, 1.0, 2.0))

    @pl.when(s >= NK1)
    def _():
        j = s - NK1
        p1_scr[:] = p1_scr[:] + _dot_nt(a1_scr[j], W11_ref[:])
        p2_scr[:] = p2_scr[:] + _dot_nt(a2_scr[j], W22_ref[:])
        p3_scr[:] = p3_scr[:] + _dot_nt(a3_scr[j], W33_ref[:])

    @pl.when(s == _S_END - 1)
    def _():
        f1 = jnp.maximum(p1_scr[:] + b11_ref[:], 0.0)
        f2 = jnp.maximum(p2_scr[:] + b22_ref[:], 0.0)
        f3 = jnp.maximum(p3_scr[:] + b33_ref[:], 0.0)
        idx = idx_scr[:]
        routed = jnp.where(idx == 0.0, f1,
                           jnp.where(idx == 1.0, f2, f3))
        o = _dot_nt(routed, Wout_ref[:])
        o = jnp.maximum(o + bout_ref[:], 0.0)
        out_ref[:] = o
        m = jnp.max(o, axis=1, keepdims=True)
        lse = jnp.log(jnp.sum(jnp.exp(o - m), axis=1, keepdims=True)) + m
        logp = o - lse
        cols = jax.lax.broadcasted_iota(jnp.int32, (BATCH, OUT), 1)
        oh = (cols == tgt_ref[:]).astype(jnp.float32)
        per_row = jnp.sum(logp * oh, axis=1, keepdims=True)
        loss_ref[:] = -jnp.sum(per_row, axis=0, keepdims=True) / BATCH


def kernel(x, target, selector_loss, W0, b0, Wsel, bsel, Wsg, bsg, Wsgo, bsgo,
           W1, b1, W11, b11, W2, b2, W22, b22, W3, b3, W33, b33, Wout, bout):
    x = x.reshape(-1, IN)
    tgt = target.reshape(BATCH, 1).astype(jnp.int32)

    const2 = lambda shp: pl.BlockSpec(shp, lambda s: (0, 0))
    w0spec = pl.BlockSpec((BK, IN), lambda s: (jnp.minimum(s, NK1 - 1), 0))
    b0spec = pl.BlockSpec((1, BK), lambda s: (0, jnp.minimum(s, NK1 - 1)))
    wselspec = pl.BlockSpec((3, BK), lambda s: (0, jnp.minimum(s, NK1 - 1)))
    w1spec = pl.BlockSpec((H2, BK), lambda s: (0, jnp.minimum(s, NK1 - 1)))
    w2spec = pl.BlockSpec(
        (H3, BK), lambda s: (0, jnp.clip(s - NK1, 0, NK2 - 1)))

    out, loss, synloss = pl.pallas_call(
        _mega_kernel,
        grid=(_S_END,),
        in_specs=[
            const2((BATCH, IN)),
            w0spec,
            b0spec,
            wselspec,
            const2((1, 3)),
            const2((1, 3)),
            const2((1, 1)),
            const2((BATCH, 1)),
            const2((1, 1)),
            const2((1, 1)),
            w1spec, w1spec, w1spec,
            const2((1, H2)), const2((1, H2)), const2((1, H2)),
            w2spec, w2spec, w2spec,
            const2((1, H3)), const2((1, H3)), const2((1, H3)),
            const2((OUT, H3)),
            const2((1, OUT)),
            const2((BATCH, 1)),
        ],
        out_specs=[
            const2((BATCH, OUT)),
            const2((1, 1)),
            const2((1, 1)),
        ],
        out_shape=[
            jax.ShapeDtypeStruct((BATCH, OUT), jnp.float32),
            jax.ShapeDtypeStruct((1, 1), jnp.float32),
            jax.ShapeDtypeStruct((1, 1), jnp.float32),
        ],
        scratch_shapes=[
            pltpu.VMEM((BATCH, 3), jnp.float32),
            pltpu.VMEM((BATCH, 1), jnp.float32),
            pltpu.VMEM((NJ, BATCH, BK), jnp.float32),
            pltpu.VMEM((NJ, BATCH, BK), jnp.float32),
            pltpu.VMEM((NJ, BATCH, BK), jnp.float32),
            pltpu.VMEM((BATCH, H3), jnp.float32),
            pltpu.VMEM((BATCH, H3), jnp.float32),
            pltpu.VMEM((BATCH, H3), jnp.float32),
        ],
    )(x, W0, b0.reshape(1, HID), Wsel, bsel.reshape(1, 3), Wsg,
      bsg.reshape(1, 1), Wsgo.reshape(BATCH, 1), bsgo.reshape(1, 1),
      selector_loss.reshape(1, 1),
      W1, W2, W3, b1.reshape(1, H2), b2.reshape(1, H2), b3.reshape(1, H2),
      W11, W22, W33, b11.reshape(1, H3), b22.reshape(1, H3),
      b33.reshape(1, H3), Wout, bout.reshape(1, OUT), tgt)

    return (out, loss[0, 0], synloss[0, 0])
